# baseline (speedup 1.0000x reference)
import jax
import jax.numpy as jnp
from jax import lax
from jax.experimental import pallas as pl
from jax.experimental.pallas import tpu as pltpu

TOP_K = 64
NHIDDEN = 16
D = 512
N = 1194
GROUPS = 6
GLEN = 199
BLK_M = 256
GRID_M = 5
HP = lax.Precision.HIGHEST
_DN = (((1,), (0,)), ((), ()))
_DT = (((0,), (1,)), ((), ()))


def _mlp_copies(W1_hbm, b1_hbm, W2_hbm, b2_hbm, W1v, b1v, W2v, b2v, sems):
    return (pltpu.make_async_copy(W1_hbm, W1v, sems.at[1]),
            pltpu.make_async_copy(b1_hbm, b1v, sems.at[2]),
            pltpu.make_async_copy(W2_hbm, W2v, sems.at[3]),
            pltpu.make_async_copy(b2_hbm, b2v, sems.at[4]))


def _body(d1_ref, d2_ref, W_hbm, W1_hbm, b1_hbm, W2_hbm, b2_hbm, out_ref,
          s2_ref, Wf, Wh_ref, W1v, b1v, W2v, b2v, sems):
    f32 = jnp.float32
    bf16 = jnp.bfloat16
    i = pl.program_id(0)

    @pl.when(i == 0)
    def _stage_weights():
        w_cp = pltpu.make_async_copy(W_hbm, Wf, sems.at[0])
        w_cp.start()
        for cp in _mlp_copies(W1_hbm, b1_hbm, W2_hbm, b2_hbm,
                              W1v, b1v, W2v, b2v, sems):
            cp.start()
        w_cp.wait()
        Wh_ref[...] = Wf[...].astype(bf16)

    Wh = Wh_ref[...]
    E1 = lax.dot_general(Wh, d1_ref[...].astype(bf16), _DT,
                         preferred_element_type=f32)
    E2 = lax.dot_general(Wh, d2_ref[...].astype(bf16), _DT,
                         preferred_element_type=f32)
    Ee = (E1 - E2) + 1e-6
    s2_ref[pl.ds(i, 1), :] = jnp.sum(Ee * Ee, axis=0, keepdims=True)

    @pl.when(i == GRID_M - 1)
    def _epilogue():
        i32 = jnp.int32
        s2pad = jnp.concatenate(
            [s2_ref[j:j + 1, :] for j in range(GRID_M)], axis=1)
        s2row = s2pad[:, :N]

        v6 = jnp.concatenate(
            [s2row[:, g * GLEN:(g + 1) * GLEN] for g in range(GROUPS)],
            axis=0)
        eye = (lax.broadcasted_iota(i32, (GLEN, GLEN), 0)
               == lax.broadcasted_iota(i32, (GLEN, GLEN), 1)).astype(f32)
        colall = lax.dot_general(
            eye, v6, dimension_numbers=(((1,), (1,)), ((), ())),
            precision=HP, preferred_element_type=f32)

        gio = lax.broadcasted_iota(i32, (GROUPS, N), 0)
        tio = lax.broadcasted_iota(i32, (GROUPS, N), 1)
        expander = (tio // GLEN == gio).astype(f32)
        col_side = lax.dot_general(
            colall, expander, _DN, precision=HP,
            preferred_element_type=f32)

        jrow = lax.broadcasted_iota(i32, (1, N), 1) % GLEN
        icol = lax.broadcasted_iota(i32, (GLEN, 1), 0)
        gt = s2row > col_side
        tie = (s2row == col_side) & (jrow < icol)
        cnt = gt.astype(f32) + tie.astype(f32)

        blockones = (lax.broadcasted_iota(i32, (N, GROUPS), 0) // GLEN
                     == lax.broadcasted_iota(
                         i32, (N, GROUPS), 1)).astype(f32)
        rank_all = lax.dot_general(
            cnt, blockones, _DN, preferred_element_type=f32)

        exp64 = (lax.broadcasted_iota(i32, (GROUPS, GROUPS * TOP_K), 1)
                 // TOP_K
                 == lax.broadcasted_iota(
                     i32, (GROUPS, GROUPS * TOP_K), 0)).astype(f32)
        rank_exp = lax.dot_general(
            rank_all, exp64, _DN, preferred_element_type=f32)
        kio = (lax.broadcasted_iota(i32, (1, GROUPS * TOP_K), 1)
               % TOP_K).astype(f32)
        oh = (rank_exp == kio).astype(bf16)

        v6h = v6.astype(bf16)
        v6l = (v6 - v6h.astype(f32)).astype(bf16)
        X = (lax.dot_general(v6h, oh, _DN, preferred_element_type=f32)
             + lax.dot_general(v6l, oh, _DN, preferred_element_type=f32))
        gmask = (lax.broadcasted_iota(i32, (GROUPS, GROUPS * TOP_K), 1)
                 // TOP_K
                 == lax.broadcasted_iota(
                     i32, (GROUPS, GROUPS * TOP_K), 0)).astype(f32)
        Xm = X * gmask
        xs = Xm[:, 0:TOP_K]
        for b in range(1, GROUPS):
            xs = xs + Xm[:, b * TOP_K:(b + 1) * TOP_K]

        x = jnp.sqrt(xs)
        for cp in _mlp_copies(W1_hbm, b1_hbm, W2_hbm, b2_hbm,
                              W1v, b1v, W2v, b2v, sems):
            cp.wait()
        h = jnp.maximum(
            lax.dot_general(x.astype(bf16), W1v[...].astype(bf16), _DN,
                            preferred_element_type=f32) + b1v[...], 0.0)
        out_ref[...] = (
            lax.dot_general(h.astype(bf16), W2v[...].astype(bf16), _DN,
                            preferred_element_type=f32) + b2v[...])


def kernel(data1, data2, W_emb, b_emb, W1, b1, W2, b2):
    del b_emb
    out = pl.pallas_call(
        _body,
        grid=(GRID_M,),
        in_specs=[
            pl.BlockSpec((BLK_M, D), lambda i: (i, 0)),
            pl.BlockSpec((BLK_M, D), lambda i: (i, 0)),
            pl.BlockSpec(memory_space=pl.ANY),
            pl.BlockSpec(memory_space=pl.ANY),
            pl.BlockSpec(memory_space=pl.ANY),
            pl.BlockSpec(memory_space=pl.ANY),
            pl.BlockSpec(memory_space=pl.ANY),
        ],
        out_specs=pl.BlockSpec((GROUPS, 1), lambda i: (0, 0)),
        out_shape=jax.ShapeDtypeStruct((GROUPS, 1), jnp.float32),
        scratch_shapes=[
            pltpu.VMEM((GRID_M, BLK_M), jnp.float32),
            pltpu.VMEM((D, D), jnp.float32),
            pltpu.VMEM((D, D), jnp.bfloat16),
            pltpu.VMEM((TOP_K, NHIDDEN), jnp.float32),
            pltpu.VMEM((1, NHIDDEN), jnp.float32),
            pltpu.VMEM((NHIDDEN, 1), jnp.float32),
            pltpu.VMEM((1, 1), jnp.float32),
            pltpu.SemaphoreType.DMA((5,)),
        ],
    )(data1, data2, W_emb, W1, b1.reshape(1, NHIDDEN), W2, b2.reshape(1, 1))
    return out


# speedup vs baseline: 1.1946x; 1.1946x over previous
import jax
import jax.numpy as jnp
from jax import lax
from jax.experimental import pallas as pl
from jax.experimental.pallas import tpu as pltpu

TOP_K = 64
NHIDDEN = 16
D = 512
N = 1194
GROUPS = 6
GLEN = 199
BLK_M = 256
GRID_M = 5
HP = lax.Precision.HIGHEST
_DN = (((1,), (0,)), ((), ()))
_DT = (((0,), (1,)), ((), ()))


def _body(d1_ref, d2_ref, W_ref, W1_ref, b1_ref, W2_ref, b2_ref, out_ref):
    f32 = jnp.float32
    bf16 = jnp.bfloat16
    i32 = jnp.int32

    Wh = W_ref[...].astype(bf16)
    E1 = lax.dot_general(Wh, d1_ref[...].astype(bf16), _DT,
                         preferred_element_type=f32)
    E2 = lax.dot_general(Wh, d2_ref[...].astype(bf16), _DT,
                         preferred_element_type=f32)
    Ee = (E1 - E2) + 1e-6
    s2row = jnp.sum(Ee * Ee, axis=0, keepdims=True)

    if True:

        v6 = jnp.concatenate(
            [s2row[:, g * GLEN:(g + 1) * GLEN] for g in range(GROUPS)],
            axis=0)
        eye = (lax.broadcasted_iota(i32, (GLEN, GLEN), 0)
               == lax.broadcasted_iota(i32, (GLEN, GLEN), 1)).astype(f32)
        colall = lax.dot_general(
            eye, v6, dimension_numbers=(((1,), (1,)), ((), ())),
            precision=HP, preferred_element_type=f32)

        gio = lax.broadcasted_iota(i32, (GROUPS, N), 0)
        tio = lax.broadcasted_iota(i32, (GROUPS, N), 1)
        expander = (tio // GLEN == gio).astype(f32)
        col_side = lax.dot_general(
            colall, expander, _DN, precision=HP,
            preferred_element_type=f32)

        jrow = lax.broadcasted_iota(i32, (1, N), 1) % GLEN
        icol = lax.broadcasted_iota(i32, (GLEN, 1), 0)
        gt = s2row > col_side
        tie = (s2row == col_side) & (jrow < icol)
        cnt = gt.astype(f32) + tie.astype(f32)

        blockones = (lax.broadcasted_iota(i32, (N, GROUPS), 0) // GLEN
                     == lax.broadcasted_iota(
                         i32, (N, GROUPS), 1)).astype(f32)
        rank_all = lax.dot_general(
            cnt, blockones, _DN, preferred_element_type=f32)

        exp64 = (lax.broadcasted_iota(i32, (GROUPS, GROUPS * TOP_K), 1)
                 // TOP_K
                 == lax.broadcasted_iota(
                     i32, (GROUPS, GROUPS * TOP_K), 0)).astype(f32)
        rank_exp = lax.dot_general(
            rank_all, exp64, _DN, preferred_element_type=f32)
        kio = (lax.broadcasted_iota(i32, (1, GROUPS * TOP_K), 1)
               % TOP_K).astype(f32)
        oh = (rank_exp == kio).astype(bf16)

        v6h = v6.astype(bf16)
        v6l = (v6 - v6h.astype(f32)).astype(bf16)
        X = (lax.dot_general(v6h, oh, _DN, preferred_element_type=f32)
             + lax.dot_general(v6l, oh, _DN, preferred_element_type=f32))
        gmask = (lax.broadcasted_iota(i32, (GROUPS, GROUPS * TOP_K), 1)
                 // TOP_K
                 == lax.broadcasted_iota(
                     i32, (GROUPS, GROUPS * TOP_K), 0)).astype(f32)
        Xm = X * gmask
        xs = Xm[:, 0:TOP_K]
        for b in range(1, GROUPS):
            xs = xs + Xm[:, b * TOP_K:(b + 1) * TOP_K]

        x = jnp.sqrt(xs)
        h = jnp.maximum(
            lax.dot_general(x.astype(bf16), W1_ref[...].astype(bf16), _DN,
                            preferred_element_type=f32) + b1_ref[...], 0.0)
        out_ref[...] = (
            lax.dot_general(h.astype(bf16), W2_ref[...].astype(bf16), _DN,
                            preferred_element_type=f32) + b2_ref[...])


def kernel(data1, data2, W_emb, b_emb, W1, b1, W2, b2):
    del b_emb
    out = pl.pallas_call(
        _body,
        out_shape=jax.ShapeDtypeStruct((GROUPS, 1), jnp.float32),
    )(data1, data2, W_emb, W1, b1.reshape(1, NHIDDEN), W2, b2.reshape(1, 1))
    return out


# speedup vs baseline: 1.2849x; 1.0756x over previous
import jax
import jax.numpy as jnp
from jax import lax
from jax.experimental import pallas as pl

TOP_K = 64
NHIDDEN = 16
D = 512
N = 1194
GROUPS = 6
GLEN = 199
_DN = (((1,), (0,)), ((), ()))
_DT = (((0,), (1,)), ((), ()))
_DC = (((1,), (1,)), ((), ()))


def _split3(x):
    bf16 = jnp.bfloat16
    f32 = jnp.float32
    hi = x.astype(bf16)
    r1 = x - hi.astype(f32)
    mid = r1.astype(bf16)
    lo = (r1 - mid.astype(f32)).astype(bf16)
    return hi, mid, lo


def _body(d1_ref, d2_ref, W_ref, W1_ref, b1_ref, W2_ref, b2_ref, out_ref):
    f32 = jnp.float32
    bf16 = jnp.bfloat16
    i32 = jnp.int32

    Wh = W_ref[...].astype(bf16)
    E1 = lax.dot_general(Wh, d1_ref[...].astype(bf16), _DT,
                         preferred_element_type=f32)
    E2 = lax.dot_general(Wh, d2_ref[...].astype(bf16), _DT,
                         preferred_element_type=f32)
    Ee = (E1 - E2) + 1e-6
    s2row = jnp.sum(Ee * Ee, axis=0, keepdims=True)

    v6 = jnp.concatenate(
        [s2row[:, g * GLEN:(g + 1) * GLEN] for g in range(GROUPS)], axis=0)

    eyeb = (lax.broadcasted_iota(i32, (GLEN, GLEN), 0)
            == lax.broadcasted_iota(i32, (GLEN, GLEN), 1)).astype(bf16)
    v6h, v6m, v6l = _split3(v6)
    colall = (lax.dot_general(eyeb, v6h, _DC, preferred_element_type=f32)
              + (lax.dot_general(eyeb, v6m, _DC, preferred_element_type=f32)
                 + lax.dot_general(eyeb, v6l, _DC,
                                   preferred_element_type=f32)))

    lanio = lax.broadcasted_iota(i32, (1, GLEN), 1)
    subio = lax.broadcasted_iota(i32, (GLEN, 1), 0)
    jlti = lanio < subio
    cnts = []
    for g in range(GROUPS):
        rowg = v6[g:g + 1, :]
        colg = colall[:, g:g + 1]
        gt = (rowg > colg).astype(f32)
        tie = ((rowg == colg) & jlti).astype(f32)
        cnts.append((gt + tie).astype(bf16))
    cnt = jnp.concatenate(cnts, axis=1)

    blockones = jnp.concatenate(
        [jnp.broadcast_to(
            (lax.broadcasted_iota(i32, (1, GROUPS), 1) == g).astype(f32),
            (GLEN, GROUPS))
         for g in range(GROUPS)], axis=0).astype(bf16)
    rank_all = lax.dot_general(
        cnt, blockones, _DN, preferred_element_type=f32)

    exp64 = jnp.concatenate(
        [jnp.broadcast_to(
            (lax.broadcasted_iota(i32, (GROUPS, 1), 0) == b).astype(f32),
            (GROUPS, TOP_K))
         for b in range(GROUPS)], axis=1).astype(bf16)
    rank_exp = lax.dot_general(
        rank_all.astype(bf16), exp64, _DN,
        preferred_element_type=f32)
    kio = jnp.concatenate(
        [lax.broadcasted_iota(i32, (1, TOP_K), 1)] * GROUPS,
        axis=1).astype(f32)
    oh = (rank_exp == kio).astype(bf16)

    X = (lax.dot_general(v6h, oh, _DN, preferred_element_type=f32)
         + (lax.dot_general(v6m, oh, _DN, preferred_element_type=f32)
            + lax.dot_general(v6l, oh, _DN, preferred_element_type=f32)))
    gmaskf = jnp.concatenate(
        [jnp.broadcast_to(
            (lax.broadcasted_iota(i32, (GROUPS, 1), 0) == b).astype(f32),
            (GROUPS, TOP_K))
         for b in range(GROUPS)], axis=1)
    Xm = X * gmaskf
    xs = Xm[:, 0:TOP_K]
    for b in range(1, GROUPS):
        xs = xs + Xm[:, b * TOP_K:(b + 1) * TOP_K]

    x = jnp.sqrt(xs)
    h = jnp.maximum(
        lax.dot_general(x.astype(bf16), W1_ref[...].astype(bf16), _DN,
                        preferred_element_type=f32) + b1_ref[...], 0.0)
    out_ref[...] = (
        lax.dot_general(h.astype(bf16), W2_ref[...].astype(bf16), _DN,
                        preferred_element_type=f32) + b2_ref[...])


def kernel(data1, data2, W_emb, b_emb, W1, b1, W2, b2):
    del b_emb
    out = pl.pallas_call(
        _body,
        out_shape=jax.ShapeDtypeStruct((GROUPS, 1), jnp.float32),
    )(data1, data2, W_emb, W1, b1.reshape(1, NHIDDEN), W2, b2.reshape(1, 1))
    return out
